# baseline (speedup 1.0000x reference)
import functools

import jax
import jax.numpy as jnp
from jax import lax
from jax.experimental import pallas as pl
from jax.experimental.pallas import tpu as pltpu
from jax.experimental.pallas import tpu_sc as plsc

BS = 32
P = 196
D = 32
K = 512
PS = 16
NROW = 14
IMG = 224
PPAD = 224
RPI = IMG * NROW
HCH = PPAD // 2


def _assign_body(x_ref, c_ref, o_ref):
    c = c_ref[...]
    cn = jnp.sum(c * c, axis=0)
    x = x_ref[0]
    s = cn[None, :] - 2.0 * lax.dot_general(
        x, c, (((1,), (0,)), ((), ())),
        preferred_element_type=jnp.float32,
        precision=lax.Precision.HIGHEST)
    o_ref[0, 0, :] = jnp.argmax(s, axis=1).astype(jnp.int32)


def _transpose_body(l_ref, o_ref):
    o_ref[...] = l_ref[...].T


def _sc_gather_body(a_hbm, t_hbm, out_hbm, a_v, patches_v, stage_v, sem):
    wid = lax.axis_index("s") * 2 + lax.axis_index("c")
    pltpu.sync_copy(a_hbm.at[wid], a_v)
    cp0 = pltpu.async_copy(t_hbm.at[a_v.at[pl.ds(0, HCH)]],
                           patches_v.at[pl.ds(0, HCH)], sem)
    cp1 = pltpu.async_copy(t_hbm.at[a_v.at[pl.ds(HCH, HCH)]],
                           patches_v.at[pl.ds(HCH, HCH)], sem)
    cp0.wait()
    cp1.wait()

    def row_body(r, carry):
        pr = r * NROW
        rb = r * PS
        for c in range(NROW):
            for i in range(PS):
                stage_v[(rb + i) * NROW + c] = \
                    patches_v[pr + c, pl.ds(i * PS, PS)]
        return carry

    lax.fori_loop(0, NROW, row_body, 0)
    pltpu.sync_copy(stage_v, out_hbm.at[wid])


@functools.cache
def _sc_gather():
    return pl.kernel(
        _sc_gather_body,
        out_type=jax.ShapeDtypeStruct((BS, RPI, PS), jnp.int32),
        mesh=plsc.VectorSubcoreMesh(core_axis_name="c", subcore_axis_name="s"),
        compiler_params=pltpu.CompilerParams(use_tc_tiling_on_sc=False),
        scratch_types=[
            pltpu.VMEM((PPAD,), jnp.int32),
            pltpu.VMEM((PPAD, PS * PS), jnp.int32),
            pltpu.VMEM((RPI, PS), jnp.int32),
            pltpu.SemaphoreType.DMA,
        ],
    )


def kernel(image, centroids, cluster_labels):
    img_p = jnp.pad(image, ((0, 0), (0, PPAD - P), (0, 0)))
    assign = pl.pallas_call(
        _assign_body,
        grid=(BS,),
        in_specs=[
            pl.BlockSpec((1, PPAD, D), lambda b: (b, 0, 0)),
            pl.BlockSpec((D, K), lambda b: (0, 0)),
        ],
        out_specs=pl.BlockSpec((1, 1, PPAD), lambda b: (b, 0, 0)),
        out_shape=jax.ShapeDtypeStruct((BS, 1, PPAD), jnp.int32),
    )(img_p, centroids)
    labels_t = pl.pallas_call(
        _transpose_body,
        out_shape=jax.ShapeDtypeStruct((K, PS * PS), jnp.int32),
    )(cluster_labels)
    out = _sc_gather()(assign.reshape(BS, PPAD), labels_t)
    return out.reshape(BS, IMG, IMG)


# speedup vs baseline: 1.5730x; 1.5730x over previous
import functools

import jax
import jax.numpy as jnp
from jax import lax
from jax.experimental import pallas as pl
from jax.experimental.pallas import tpu as pltpu
from jax.experimental.pallas import tpu_sc as plsc

BS = 32
P = 196
D = 32
K = 512
PS = 16
NROW = 14
IMG = 224
PPAD = 224
RPI = IMG * NROW
IDX_PAD = RPI + PS
NCHUNK = 28
CH = RPI // NCHUNK


def _assign_body(x_ref, c_ref, o_ref):
    c = c_ref[...]
    cn = jnp.sum(c * c, axis=0)
    x = x_ref[0]
    s = cn[None, :] - 2.0 * lax.dot_general(
        x, c, (((1,), (0,)), ((), ())),
        preferred_element_type=jnp.float32,
        precision=lax.Precision.HIGHEST)
    amax = jnp.argmax(s, axis=1).astype(jnp.int32)
    o_ref[0] = amax[:, None] * PS + \
        lax.broadcasted_iota(jnp.int32, (PPAD, PS), 1)


def _transpose_body(l_ref, o_ref):
    o_ref[...] = l_ref[...].T


def _sc_gather_body(a_hbm, t_hbm, out_hbm, a_v, idx_v, rows_v, sem, sem1):
    wid = lax.axis_index("s") * 2 + lax.axis_index("c")
    pltpu.sync_copy(a_hbm.at[wid], a_v)
    for c in range(NROW):
        for r in range(NROW):
            idx_v[pl.ds(c * IMG + r * PS, PS)] = \
                a_v[pl.ds((r * NROW + c) * PS, PS)]
    cps = [pltpu.async_copy(t_hbm.at[idx_v.at[pl.ds(j * CH, CH)]],
                            rows_v.at[pl.ds(j * CH, CH)], sem)
           for j in range(NCHUNK)]
    for cp in cps:
        cp.wait()
    ocps = [pltpu.async_copy(rows_v.at[pl.ds(c * IMG, IMG)],
                             out_hbm.at[wid, :, pl.ds(c * PS, PS)], sem1)
            for c in range(NROW)]
    for cp in ocps:
        cp.wait()


@functools.cache
def _sc_gather():
    return pl.kernel(
        _sc_gather_body,
        out_type=jax.ShapeDtypeStruct((BS, IMG, IMG), jnp.int32),
        mesh=plsc.VectorSubcoreMesh(core_axis_name="c", subcore_axis_name="s"),
        compiler_params=pltpu.CompilerParams(use_tc_tiling_on_sc=False),
        scratch_types=[
            pltpu.VMEM((PPAD * PS,), jnp.int32),
            pltpu.VMEM((RPI,), jnp.int32),
            pltpu.VMEM((RPI, PS), jnp.int32),
            pltpu.SemaphoreType.DMA,
            pltpu.SemaphoreType.DMA,
        ],
    )


def kernel(image, centroids, cluster_labels):
    img_p = jnp.pad(image, ((0, 0), (0, PPAD - P), (0, 0)))
    assign = pl.pallas_call(
        _assign_body,
        grid=(BS,),
        in_specs=[
            pl.BlockSpec((1, PPAD, D), lambda b: (b, 0, 0)),
            pl.BlockSpec((D, K), lambda b: (0, 0)),
        ],
        out_specs=pl.BlockSpec((1, PPAD, PS), lambda b: (b, 0, 0)),
        out_shape=jax.ShapeDtypeStruct((BS, PPAD, PS), jnp.int32),
    )(img_p, centroids)
    labels_t = pl.pallas_call(
        _transpose_body,
        out_shape=jax.ShapeDtypeStruct((K, PS * PS), jnp.int32),
    )(cluster_labels)
    return _sc_gather()(assign.reshape(BS, PPAD * PS),
                        labels_t.reshape(K * PS, PS))


# speedup vs baseline: 1.7434x; 1.1083x over previous
import functools

import jax
import jax.numpy as jnp
from jax import lax
from jax.experimental import pallas as pl
from jax.experimental.pallas import tpu as pltpu
from jax.experimental.pallas import tpu_sc as plsc

BS = 32
P = 196
D = 32
K = 512
PS = 16
NROW = 14
IMG = 224
RPI = IMG * NROW
NCHUNK = 28
CH = RPI // NCHUNK
RB = 784


def _assign_body(x_ref, c_ref, o_ref):
    c = c_ref[...]
    cn = jnp.sum(c * c, axis=0)
    xt = x_ref[0]
    s = cn[None, :] - 2.0 * lax.dot_general(
        xt, c, (((0,), (0,)), ((), ())),
        preferred_element_type=jnp.float32,
        precision=lax.Precision.HIGHEST)
    amax = jnp.argmax(s, axis=1).astype(jnp.int32)
    io = lax.broadcasted_iota(jnp.int32, (P, PS), 1)
    o_ref[0] = amax[:, None] * 8 + (io % 8) + (io // 8) * 4096


def _transpose_body(l_ref, o_ref):
    o_ref[...] = l_ref[...].T


def _sc_gather_body(a_hbm, t_hbm, out_hbm, a_v, idx_v, rows_v, sem, sem1):
    wid = lax.axis_index("s") * 2 + lax.axis_index("c")
    pltpu.sync_copy(a_hbm.at[wid], a_v)
    for c in range(NROW):
        for r in range(NROW):
            idx_v[pl.ds(c * IMG + r * PS, PS)] = \
                a_v[pl.ds((r * NROW + c) * PS, PS)]
    cps = [pltpu.async_copy(t_hbm.at[idx_v.at[pl.ds(j * CH, CH)]],
                            rows_v.at[pl.ds(j * CH, CH)], sem)
           for j in range(NCHUNK)]
    for cp in cps:
        cp.wait()
    ocps = [pltpu.async_copy(rows_v.at[pl.ds(c * IMG, IMG)],
                             out_hbm.at[wid, :, pl.ds(c * PS, PS)], sem1)
            for c in range(NROW)]
    for cp in ocps:
        cp.wait()


@functools.cache
def _sc_gather():
    return pl.kernel(
        _sc_gather_body,
        out_type=jax.ShapeDtypeStruct((BS, IMG, IMG), jnp.int32),
        mesh=plsc.VectorSubcoreMesh(core_axis_name="c", subcore_axis_name="s"),
        compiler_params=pltpu.CompilerParams(use_tc_tiling_on_sc=False),
        scratch_types=[
            pltpu.VMEM((P * PS,), jnp.int32),
            pltpu.VMEM((RPI,), jnp.int32),
            pltpu.VMEM((RPI, PS), jnp.int32),
            pltpu.SemaphoreType.DMA,
            pltpu.SemaphoreType.DMA,
        ],
    )


def kernel(image, centroids, cluster_labels):
    assign = pl.pallas_call(
        _assign_body,
        grid=(BS,),
        in_specs=[
            pl.BlockSpec((1, D, P), lambda b: (b, 0, 0)),
            pl.BlockSpec((D, K), lambda b: (0, 0)),
        ],
        out_specs=pl.BlockSpec((1, P, PS), lambda b: (b, 0, 0)),
        out_shape=jax.ShapeDtypeStruct((BS, P, PS), jnp.int32),
    )(image.transpose(0, 2, 1), centroids)
    labels_t = pl.pallas_call(
        _transpose_body,
        grid=(2,),
        in_specs=[pl.BlockSpec((K // 4, K), lambda h: (h, 0))],
        out_specs=pl.BlockSpec((K, K // 4), lambda h: (h, 0)),
        out_shape=jax.ShapeDtypeStruct((2 * K, K // 4), jnp.int32),
    )(cluster_labels)
    return _sc_gather()(assign.reshape(BS, P * PS),
                        labels_t.reshape(K * PS, PS))
